# Initial kernel scaffold; baseline (speedup 1.0000x reference)
import jax
import jax.numpy as jnp
from jax.experimental import pallas as pl


def kernel(inputs, embedding):
    raise NotImplementedError("write your pallas kernel here")


# speedup vs baseline: 3.1118x; 3.1118x over previous
import functools

import jax
import jax.numpy as jnp
from jax.experimental import pallas as pl
from jax.experimental.pallas import tpu as pltpu

NUM_EMB = 1024
EMB_DIM = 64
N_ROWS = 16 * 1024
BLOCK_ROWS = 512
N_BLOCKS = N_ROWS // BLOCK_ROWS


def _vq_kernel(x_ref, emb_ref, xsq_ref, esq_ref,
               enc_ref, qst_ref, loss_ref, perp_ref,
               loss_acc, cnt_acc):
    i = pl.program_id(0)

    @pl.when(i == 0)
    def _init():
        loss_acc[0] = 0.0
        cnt_acc[...] = jnp.zeros_like(cnt_acc)

    x = x_ref[...]
    emb = emb_ref[...]
    xsq = xsq_ref[...].reshape(BLOCK_ROWS, 1)
    esq = esq_ref[...]

    mm = jax.lax.dot_general(x, emb, (((1,), (1,)), ((), ())),
                             preferred_element_type=jnp.float32)
    d = (xsq + esq) - 2.0 * mm

    dmin = jnp.min(d, axis=1, keepdims=True)
    iota = jax.lax.broadcasted_iota(jnp.int32, (BLOCK_ROWS, NUM_EMB), 1)
    idx = jnp.min(jnp.where(d == dmin, iota, NUM_EMB), axis=1, keepdims=True)

    enc = (iota == idx).astype(jnp.float32)
    enc_ref[...] = enc

    q = jax.lax.dot_general(enc, emb, (((1,), (0,)), ((), ())),
                            preferred_element_type=jnp.float32)
    diff = q - x
    qst_ref[...] = x + diff

    loss_acc[0] += jnp.sum(diff * diff)
    cnt_acc[...] += jnp.sum(enc, axis=0, keepdims=True)

    @pl.when(i == N_BLOCKS - 1)
    def _fini():
        m = loss_acc[0] * (1.0 / (N_ROWS * EMB_DIM))
        loss_ref[...] = jnp.reshape(m + 0.25 * m, (1, 1))
        probs = cnt_acc[...] * (1.0 / N_ROWS)
        ent = jnp.sum(probs * jnp.log(probs + 1e-10))
        perp_ref[...] = jnp.reshape(jnp.exp(-ent), (1, 1))


@functools.partial(jax.jit)
def kernel(inputs, embedding):
    input_shape = inputs.shape
    flat = inputs.reshape(-1, EMB_DIM)
    xsq = jnp.sum(flat ** 2, axis=1, keepdims=True)
    esq = jnp.sum(embedding ** 2, axis=1)

    enc, qst, loss, perp = pl.pallas_call(
        _vq_kernel,
        grid=(N_BLOCKS,),
        in_specs=[
            pl.BlockSpec((BLOCK_ROWS, EMB_DIM), lambda i: (i, 0)),
            pl.BlockSpec((NUM_EMB, EMB_DIM), lambda i: (0, 0)),
            pl.BlockSpec((1, 1, BLOCK_ROWS), lambda i: (i, 0, 0)),
            pl.BlockSpec((1, NUM_EMB), lambda i: (0, 0)),
        ],
        out_specs=[
            pl.BlockSpec((BLOCK_ROWS, NUM_EMB), lambda i: (i, 0)),
            pl.BlockSpec((BLOCK_ROWS, EMB_DIM), lambda i: (i, 0)),
            pl.BlockSpec((1, 1), lambda i: (0, 0)),
            pl.BlockSpec((1, 1), lambda i: (0, 0)),
        ],
        out_shape=[
            jax.ShapeDtypeStruct((N_ROWS, NUM_EMB), jnp.float32),
            jax.ShapeDtypeStruct((N_ROWS, EMB_DIM), jnp.float32),
            jax.ShapeDtypeStruct((1, 1), jnp.float32),
            jax.ShapeDtypeStruct((1, 1), jnp.float32),
        ],
        scratch_shapes=[
            pltpu.SMEM((1,), jnp.float32),
            pltpu.VMEM((1, NUM_EMB), jnp.float32),
        ],
    )(flat, embedding, xsq.reshape(N_BLOCKS, 1, BLOCK_ROWS), esq.reshape(1, NUM_EMB))

    return (loss[0, 0], qst.reshape(input_shape), perp[0, 0], enc)


# speedup vs baseline: 3.3518x; 1.0771x over previous
import functools

import jax
import jax.numpy as jnp
from jax.experimental import pallas as pl
from jax.experimental.pallas import tpu as pltpu

NUM_EMB = 1024
EMB_DIM = 64
N_ROWS = 16 * 1024
BLOCK_ROWS = 1024
N_BLOCKS = N_ROWS // BLOCK_ROWS


def _vq_kernel(x_ref, emb_ref, xsq_ref, esq_ref,
               enc_ref, qst_ref, loss_ref, perp_ref,
               loss_acc, cnt_acc):
    i = pl.program_id(0)

    @pl.when(i == 0)
    def _init():
        loss_acc[0] = 0.0
        cnt_acc[...] = jnp.zeros_like(cnt_acc)

    x = x_ref[...]
    emb = emb_ref[...]
    xsq = xsq_ref[...].reshape(BLOCK_ROWS, 1)
    esq = esq_ref[...]

    mm2 = jax.lax.dot_general(x * -2.0, emb, (((1,), (1,)), ((), ())),
                              preferred_element_type=jnp.float32)
    d = (xsq + esq) + mm2

    dmin = jnp.min(d, axis=1, keepdims=True)
    iota = jax.lax.broadcasted_iota(jnp.int32, (BLOCK_ROWS, NUM_EMB), 1)
    idx = jnp.min(jnp.where(d == dmin, iota, NUM_EMB), axis=1, keepdims=True)

    enc = (iota == idx).astype(jnp.float32)
    enc_ref[...] = enc

    q = jax.lax.dot_general(enc, emb, (((1,), (0,)), ((), ())),
                            preferred_element_type=jnp.float32)
    diff = q - x
    qst_ref[...] = x + diff

    loss_acc[0] += jnp.sum(diff * diff)
    cnt_acc[...] += jnp.sum(enc, axis=0, keepdims=True)

    @pl.when(i == N_BLOCKS - 1)
    def _fini():
        m = loss_acc[0] * (1.0 / (N_ROWS * EMB_DIM))
        loss_ref[...] = jnp.reshape(m + 0.25 * m, (1, 1))
        probs = cnt_acc[...] * (1.0 / N_ROWS)
        ent = jnp.sum(probs * jnp.log(probs + 1e-10))
        perp_ref[...] = jnp.reshape(jnp.exp(-ent), (1, 1))


@functools.partial(jax.jit)
def kernel(inputs, embedding):
    input_shape = inputs.shape
    flat = inputs.reshape(-1, EMB_DIM)
    xsq = jnp.sum(flat ** 2, axis=1, keepdims=True)
    esq = jnp.sum(embedding ** 2, axis=1)

    enc, qst, loss, perp = pl.pallas_call(
        _vq_kernel,
        grid=(N_BLOCKS,),
        in_specs=[
            pl.BlockSpec((BLOCK_ROWS, EMB_DIM), lambda i: (i, 0)),
            pl.BlockSpec((NUM_EMB, EMB_DIM), lambda i: (0, 0)),
            pl.BlockSpec((1, 1, BLOCK_ROWS), lambda i: (i, 0, 0)),
            pl.BlockSpec((1, NUM_EMB), lambda i: (0, 0)),
        ],
        out_specs=[
            pl.BlockSpec((BLOCK_ROWS, NUM_EMB), lambda i: (i, 0)),
            pl.BlockSpec((BLOCK_ROWS, EMB_DIM), lambda i: (i, 0)),
            pl.BlockSpec((1, 1), lambda i: (0, 0)),
            pl.BlockSpec((1, 1), lambda i: (0, 0)),
        ],
        out_shape=[
            jax.ShapeDtypeStruct((N_ROWS, NUM_EMB), jnp.float32),
            jax.ShapeDtypeStruct((N_ROWS, EMB_DIM), jnp.float32),
            jax.ShapeDtypeStruct((1, 1), jnp.float32),
            jax.ShapeDtypeStruct((1, 1), jnp.float32),
        ],
        scratch_shapes=[
            pltpu.SMEM((1,), jnp.float32),
            pltpu.VMEM((1, NUM_EMB), jnp.float32),
        ],
    )(flat, embedding, xsq.reshape(N_BLOCKS, 1, BLOCK_ROWS), esq.reshape(1, NUM_EMB))

    return (loss[0, 0], qst.reshape(input_shape), perp[0, 0], enc)


# speedup vs baseline: 3.6031x; 1.0750x over previous
import functools

import jax
import jax.numpy as jnp
from jax.experimental import pallas as pl
from jax.experimental.pallas import tpu as pltpu

NUM_EMB = 1024
EMB_DIM = 64
N_ROWS = 16 * 1024
BLOCK_ROWS = 1024
N_BLOCKS = N_ROWS // BLOCK_ROWS


def _vq_kernel(x_ref, emb_ref, xsq_ref, esq_ref,
               enc_ref, qst_ref, loss_ref, perp_ref,
               loss_acc, cnt_acc):
    i = pl.program_id(0)

    @pl.when(i == 0)
    def _init():
        loss_acc[0] = 0.0
        cnt_acc[...] = jnp.zeros_like(cnt_acc)

    x = x_ref[...]
    emb = emb_ref[...]
    xsq = xsq_ref[...].reshape(BLOCK_ROWS, 1)
    esq = esq_ref[...]

    mm2 = jax.lax.dot_general(x * -2.0, emb, (((1,), (1,)), ((), ())),
                              preferred_element_type=jnp.float32)
    d = (xsq + esq) + mm2

    iota = jax.lax.broadcasted_iota(jnp.int32, (BLOCK_ROWS, NUM_EMB), 1)
    idx = jnp.argmin(d, axis=1).reshape(BLOCK_ROWS, 1).astype(jnp.int32)

    enc = (iota == idx).astype(jnp.float32)
    enc_ref[...] = enc

    q = jax.lax.dot_general(enc, emb, (((1,), (0,)), ((), ())),
                            preferred_element_type=jnp.float32)
    diff = q - x
    qst_ref[...] = x + diff

    loss_acc[0] += jnp.sum(diff * diff)
    cnt_acc[...] += jnp.sum(enc, axis=0, keepdims=True)

    @pl.when(i == N_BLOCKS - 1)
    def _fini():
        m = loss_acc[0] * (1.0 / (N_ROWS * EMB_DIM))
        loss_ref[...] = jnp.reshape(m + 0.25 * m, (1, 1))
        probs = cnt_acc[...] * (1.0 / N_ROWS)
        ent = jnp.sum(probs * jnp.log(probs + 1e-10))
        perp_ref[...] = jnp.reshape(jnp.exp(-ent), (1, 1))


@functools.partial(jax.jit)
def kernel(inputs, embedding):
    input_shape = inputs.shape
    flat = inputs.reshape(-1, EMB_DIM)
    xsq = jnp.sum(flat ** 2, axis=1, keepdims=True)
    esq = jnp.sum(embedding ** 2, axis=1)

    enc, qst, loss, perp = pl.pallas_call(
        _vq_kernel,
        grid=(N_BLOCKS,),
        in_specs=[
            pl.BlockSpec((BLOCK_ROWS, EMB_DIM), lambda i: (i, 0)),
            pl.BlockSpec((NUM_EMB, EMB_DIM), lambda i: (0, 0)),
            pl.BlockSpec((1, 1, BLOCK_ROWS), lambda i: (i, 0, 0)),
            pl.BlockSpec((1, NUM_EMB), lambda i: (0, 0)),
        ],
        out_specs=[
            pl.BlockSpec((BLOCK_ROWS, NUM_EMB), lambda i: (i, 0)),
            pl.BlockSpec((BLOCK_ROWS, EMB_DIM), lambda i: (i, 0)),
            pl.BlockSpec((1, 1), lambda i: (0, 0)),
            pl.BlockSpec((1, 1), lambda i: (0, 0)),
        ],
        out_shape=[
            jax.ShapeDtypeStruct((N_ROWS, NUM_EMB), jnp.float32),
            jax.ShapeDtypeStruct((N_ROWS, EMB_DIM), jnp.float32),
            jax.ShapeDtypeStruct((1, 1), jnp.float32),
            jax.ShapeDtypeStruct((1, 1), jnp.float32),
        ],
        scratch_shapes=[
            pltpu.SMEM((1,), jnp.float32),
            pltpu.VMEM((1, NUM_EMB), jnp.float32),
        ],
    )(flat, embedding, xsq.reshape(N_BLOCKS, 1, BLOCK_ROWS), esq.reshape(1, NUM_EMB))

    return (loss[0, 0], qst.reshape(input_shape), perp[0, 0], enc)
